# baseline (speedup 1.0000x reference)
import functools

import jax
import jax.numpy as jnp
from jax import lax
from jax.experimental import pallas as pl
from jax.experimental.pallas import tpu as pltpu
from jax.experimental.pallas import tpu_sc as plsc

_N = 10000
_D = 128
_E = 320000

_NC = 2
_NS = 16
_NW = _NC * _NS

_C = 128
_NB = 2
_EPT = 10400
_EPAD = _EPT * _NW

_SSPLIT = 5000
_TSPLIT = 5056
_TROWS = 5120
_AROWS = 5120
_DUMP = 5072
_LCAP = 3456
_RPT = _AROWS // _NS


def _mm_bias_kernel(x_ref, w_ref, b_ref, o_ref):
    o_ref[...] = (
        jnp.dot(x_ref[...], w_ref[...], preferred_element_type=jnp.float32)
        + b_ref[...]
    )


def _mm_bias(x, w, b2d):
    return pl.pallas_call(
        _mm_bias_kernel,
        out_shape=jax.ShapeDtypeStruct((x.shape[0], w.shape[1]), jnp.float32),
    )(x, w, b2d)


def _merge4(p0, p1, p2, p3):
    lo = p0 + p2
    hi = p1 + p3
    return jnp.concatenate([lo[:_TSPLIT], hi[:_N - _TSPLIT]], axis=0)


def _merge_relu_mm_kernel(p0_ref, p1_ref, p2_ref, p3_ref, w_ref, b_ref, o_ref):
    x = jnp.maximum(_merge4(p0_ref[...], p1_ref[...], p2_ref[...], p3_ref[...]),
                    0.0)
    o_ref[...] = (
        jnp.dot(x, w_ref[...], preferred_element_type=jnp.float32) + b_ref[...]
    )


def _merge_relu_mm(p, w, b2d):
    return pl.pallas_call(
        _merge_relu_mm_kernel,
        out_shape=jax.ShapeDtypeStruct((_N, _D), jnp.float32),
    )(p[0], p[1], p[2], p[3], w, b2d)


def _final_merge_kernel(p0_ref, p1_ref, p2_ref, p3_ref, o_ref):
    o_ref[...] = _merge4(p0_ref[...], p1_ref[...], p2_ref[...], p3_ref[...])


def _final_merge(p):
    return pl.pallas_call(
        _final_merge_kernel,
        out_shape=jax.ShapeDtypeStruct((_N, _D), jnp.float32),
    )(p[0], p[1], p[2], p[3])


_mesh = plsc.VectorSubcoreMesh(core_axis_name="c", subcore_axis_name="s")
_params = pltpu.CompilerParams(use_tc_tiling_on_sc=False,
                               needs_layout_passes=False)


@functools.partial(
    pl.kernel,
    out_type=(
        jax.ShapeDtypeStruct((_NW * 4 * _LCAP,), jnp.int32),
        jax.ShapeDtypeStruct((_NW * 4 * _LCAP,), jnp.int32),
        jax.ShapeDtypeStruct((_NW, 4, 16), jnp.int32),
    ),
    mesh=_mesh,
    compiler_params=_params,
    scratch_types=[
        pltpu.VMEM((_EPT,), jnp.int32),
        pltpu.VMEM((_EPT,), jnp.int32),
        pltpu.VMEM((4 * _LCAP + 16,), jnp.int32),
        pltpu.VMEM((4 * _LCAP + 16,), jnp.int32),
        pltpu.VMEM((4, 16), jnp.int32),
    ],
)
def _partition(src_hbm, dst_hbm, sl_hbm, dl_hbm, cnt_hbm,
               sstg, dstg, slst, dlst, cstg):
    cid = lax.axis_index("c")
    sid = lax.axis_index("s")
    wid = sid * _NC + cid
    pltpu.sync_copy(src_hbm.at[pl.ds(wid * _EPT, _EPT)], sstg)
    pltpu.sync_copy(dst_hbm.at[pl.ds(wid * _EPT, _EPT)], dstg)

    lane = lax.iota(jnp.int32, 16)

    def body(i, cnts):
        sv = sstg[pl.ds(i * 16, 16)]
        dv = dstg[pl.ds(i * 16, 16)]
        sbit = sv >= _SSPLIT
        tbit = dv >= _TSPLIT
        sl = sv - jnp.where(sbit, _SSPLIT, 0).astype(jnp.int32)
        dl = dv - jnp.where(tbit, _TSPLIT, 0).astype(jnp.int32)
        out = []
        for bkt in range(4):
            ms = sbit if bkt >= 2 else jnp.logical_not(sbit)
            mt = tbit if bkt & 1 else jnp.logical_not(tbit)
            m = jnp.logical_and(ms, mt)
            mi = m.astype(jnp.int32)
            ex = jnp.cumsum(mi) - mi
            cb = cnts[bkt]
            pos = jnp.where(m, bkt * _LCAP + cb + ex, 4 * _LCAP + lane)
            plsc.store_scatter(slst, [pos], sl)
            plsc.store_scatter(dlst, [pos], dl)
            out.append(cb + plsc.all_reduce_population_count(m)[0])
        return tuple(out)

    cnts = lax.fori_loop(0, _EPT // 16, body, (0, 0, 0, 0))

    dump_s = jnp.zeros((16,), jnp.int32)
    dump_d = jnp.full((16,), _DUMP, jnp.int32)
    for bkt in range(4):
        cb = cnts[bkt]
        pc = ((cb + 255) // 256) * 256

        def pad_body(k, _, bkt=bkt, cb=cb):
            slst[pl.ds(bkt * _LCAP + cb + k * 16, 16)] = dump_s
            dlst[pl.ds(bkt * _LCAP + cb + k * 16, 16)] = dump_d
            return _

        lax.fori_loop(0, (pc - cb + 15) // 16, pad_body, 0)
        cstg[bkt, :] = jnp.full((16,), pc, jnp.int32)

    pltpu.sync_copy(slst.at[pl.ds(0, 4 * _LCAP)],
                    sl_hbm.at[pl.ds(wid * 4 * _LCAP, 4 * _LCAP)])
    pltpu.sync_copy(dlst.at[pl.ds(0, 4 * _LCAP)],
                    dl_hbm.at[pl.ds(wid * 4 * _LCAP, 4 * _LCAP)])
    pltpu.sync_copy(cstg, cnt_hbm.at[wid])


@functools.partial(
    pl.kernel,
    out_type=jax.ShapeDtypeStruct((_NC * 2 * _AROWS, _D), jnp.float32),
    mesh=_mesh,
    compiler_params=_params,
    scratch_types=[
        pltpu.VMEM((_NB, _C), jnp.int32),
        pltpu.VMEM((_NB, _C), jnp.int32),
        pltpu.VMEM((_NB, _C, _D), jnp.float32),
        pltpu.VMEM((_NW, 4, 16), jnp.int32),
        pltpu.VMEM_SHARED((_TROWS, _D), jnp.float32),
        pltpu.VMEM_SHARED((_AROWS, _D), jnp.float32),
        pltpu.SemaphoreType.DMA((_NB,)),
        pltpu.SemaphoreType.DMA((_NB,)),
        pltpu.SemaphoreType.DMA((_NB,)),
    ],
)
def _aggregate(h_hbm, sl_hbm, dl_hbm, cnt_hbm, z_hbm, out_hbm,
               sidx, didx, rows, cnts, table, acc, isem, gsem, ssem):
    cid = lax.axis_index("c")
    sid = lax.axis_index("s")

    @pl.when(sid < 15)
    def _():
        pltpu.sync_copy(h_hbm.at[pl.ds(cid * _SSPLIT + sid * 320, 320)],
                        table.at[pl.ds(sid * 320, 320)])

    @pl.when(sid == 15)
    def _():
        pltpu.sync_copy(h_hbm.at[pl.ds(cid * _SSPLIT + 4800, 200)],
                        table.at[pl.ds(4800, 200)])

    pltpu.sync_copy(cnt_hbm, cnts)

    def iload(base, j, b):
        pltpu.async_copy(sl_hbm.at[pl.ds(base + j * _C, _C)], sidx.at[b],
                         isem.at[b])
        pltpu.async_copy(dl_hbm.at[pl.ds(base + j * _C, _C)], didx.at[b],
                         isem.at[b])

    def run_list(base, nchunk):
        grps = nchunk // 2

        @pl.when(grps > 0)
        def _():
            for b in range(_NB):
                iload(base, b, b)

            def group(g, carry):
                gath = []
                for b in range(_NB):
                    j = g * _NB + b
                    pltpu.make_async_copy(
                        sl_hbm.at[pl.ds(base + j * _C, _C)], sidx.at[b],
                        isem.at[b]).wait()
                    pltpu.make_async_copy(
                        dl_hbm.at[pl.ds(base + j * _C, _C)], didx.at[b],
                        isem.at[b]).wait()
                    gath.append(pltpu.async_copy(
                        table.at[sidx.at[b]], rows.at[b], gsem.at[b]))
                scat = []
                for b in range(_NB):
                    gath[b].wait()
                    scat.append(pltpu.async_copy(
                        rows.at[b], acc.at[didx.at[b]], ssem.at[b], add=True))
                for b in range(_NB):
                    scat[b].wait()

                    @pl.when(g + 1 < grps)
                    def _(g=g, b=b):
                        iload(base, (g + 1) * _NB + b, b)
                return carry

            lax.fori_loop(0, grps, group, 0)

    for t in (0, 1):
        pltpu.sync_copy(z_hbm, acc.at[pl.ds(sid * _RPT, _RPT)])
        plsc.subcore_barrier()
        bkt = 2 * cid + t
        for stk in (0, 1):
            stile = sid + 16 * stk
            base = (stile * 4) * _LCAP + bkt * _LCAP
            nchunk = cnts[stile, bkt][0] // _C
            run_list(base, nchunk)
        plsc.subcore_barrier()
        pltpu.sync_copy(
            acc.at[pl.ds(sid * _RPT, _RPT)],
            out_hbm.at[pl.ds((cid * 2 + t) * _AROWS + sid * _RPT, _RPT)],
        )


def kernel(A_a, X_a, Wr, br, W1, b1, W2, b2):
    del Wr, br
    ept_real = _E // _NW
    padt = _EPT - ept_real
    srcp = jnp.concatenate(
        [A_a[0].reshape(_NW, ept_real),
         jnp.zeros((_NW, padt), jnp.int32)], axis=1).reshape(-1)
    dstp = jnp.concatenate(
        [A_a[1].reshape(_NW, ept_real),
         jnp.full((_NW, padt), _N, jnp.int32)], axis=1).reshape(-1)
    zrows = jnp.zeros((_RPT, _D), jnp.float32)

    sl, dl, cnt = _partition(srcp, dstp)

    h1 = _mm_bias(X_a, W1, b1.reshape(1, _D))
    p = _aggregate(h1, sl, dl, cnt, zrows)
    planes = [p[i * _AROWS:(i + 1) * _AROWS] for i in range(4)]
    h2 = _merge_relu_mm(planes, W2, b2.reshape(1, _D))
    q = _aggregate(h2, sl, dl, cnt, zrows)
    qplanes = [q[i * _AROWS:(i + 1) * _AROWS] for i in range(4)]
    return _final_merge(qplanes)
